# Initial kernel scaffold; baseline (speedup 1.0000x reference)
import functools

import jax
import jax.numpy as jnp
import numpy as np
from jax import lax
from jax.experimental import pallas as pl
from jax.experimental.pallas import tpu as pltpu
from jax.experimental.pallas import tpu_sc as plsc

_B = 4096
_NBR = 197
_NEI = 3
_LOG_2PI = float(np.log(2.0 * np.pi))

_L = 16
_NC, _NS = 2, 16
_NW = _NC * _NS
_TPW = _B // _NW
_T = 8
_NCH = _TPW // _T
_P = _T * _NBR
_PPAD = _P + _L
_N3 = _P * _NEI
_K = (_N3 + 127) // 128
_N3PAD = _K * 128

_NVR = (_NBR + _L - 1) // _L


def _body(idx_hbm, eps_hbm, w_hbm, samp_hbm, logq_hbm,
          idx_v, rows_v, eps_v, samp_v, logq_v, sem):
  c = lax.axis_index("c")
  s = lax.axis_index("s")
  wid = s * _NC + c

  iot = lax.iota(jnp.int32, _L)
  col0 = jnp.zeros((_L,), jnp.int32)
  col1 = jnp.ones((_L,), jnp.int32)

  def chunk(ch, carry):
    tree0 = wid * _TPW + ch * _T
    pbase = tree0 * _NBR
    ibase = pbase * _NEI

    idx_v[pl.ds(_N3PAD - _L, _L)] = jnp.zeros((_L,), jnp.int32)
    pltpu.sync_copy(idx_hbm.at[pl.ds(ibase, _N3)], idx_v.at[pl.ds(0, _N3)])
    pltpu.sync_copy(eps_hbm.at[pl.ds(pbase, _P)], eps_v.at[pl.ds(0, _P)])

    def fire(j, cc):
      pltpu.async_copy(w_hbm.at[idx_v.at[pl.ds(j * 128, 128)]],
                       rows_v.at[pl.ds(j * 128, 128)], sem)
      return cc
    lax.fori_loop(0, _K, fire, 0)
    pltpu.make_async_copy(w_hbm.at[pl.ds(0, _N3PAD)], rows_v, sem).wait()

    def tree(t, cc):
      base = t * _NBR
      acc = jnp.zeros((_L,), jnp.float32)
      for j in range(_NVR):
        off = base + j * _L
        pos = jnp.minimum(off + iot, _P - 1)
        r0 = pos * _NEI
        r1 = r0 + 1
        r2 = r0 + 2
        m = (plsc.load_gather(rows_v, [r0, col0])
             + plsc.load_gather(rows_v, [r1, col0])
             + plsc.load_gather(rows_v, [r2, col0]))
        sd = (plsc.load_gather(rows_v, [r0, col1])
              + plsc.load_gather(rows_v, [r1, col1])
              + plsc.load_gather(rows_v, [r2, col1]))
        e = eps_v[pl.ds(off, _L)]
        samp_v[pl.ds(off, _L)] = e * jnp.exp(sd) + m
        valid = iot < (_NBR - j * _L)
        acc = acc + jnp.where(valid, -0.5 * e * e - sd, 0.0)
      logq_v[ch * _T + t] = jnp.sum(acc) - 0.5 * _NBR * _LOG_2PI
      return cc
    lax.fori_loop(0, _T, tree, 0)

    pltpu.sync_copy(samp_v.at[pl.ds(0, _P)], samp_hbm.at[pl.ds(pbase, _P)])
    return carry

  lax.fori_loop(0, _NCH, chunk, 0)
  pltpu.sync_copy(logq_v, logq_hbm.at[pl.ds(wid * _TPW, _TPW)])


_encoder = functools.partial(
    pl.kernel,
    out_type=[jax.ShapeDtypeStruct((_B * _NBR,), jnp.float32),
              jax.ShapeDtypeStruct((_B,), jnp.float32)],
    mesh=plsc.VectorSubcoreMesh(core_axis_name="c", subcore_axis_name="s"),
    scratch_types=[
        pltpu.VMEM((_N3PAD,), jnp.int32),
        pltpu.VMEM((_N3PAD, 2), jnp.float32),
        pltpu.VMEM((_PPAD,), jnp.float32),
        pltpu.VMEM((_PPAD,), jnp.float32),
        pltpu.VMEM((_TPW,), jnp.float32),
        pltpu.SemaphoreType.DMA,
    ],
)(_body)


@jax.jit
def kernel(neigh_ss_idxes, eps, W):
  idx_flat = neigh_ss_idxes.reshape(-1)
  eps_flat = eps.reshape(-1)
  samp_flat, logq = _encoder(idx_flat, eps_flat, W)
  return samp_flat.reshape(_B, _NBR), logq, neigh_ss_idxes


# speedup vs baseline: 10.8462x; 10.8462x over previous
import functools

import jax
import jax.numpy as jnp
import numpy as np
from jax import lax
from jax.experimental import pallas as pl
from jax.experimental.pallas import tpu as pltpu
from jax.experimental.pallas import tpu_sc as plsc

_B = 4096
_NBR = 197
_NEI = 3
_LOG_2PI = float(np.log(2.0 * np.pi))

_L = 16
_NC, _NS = 2, 16
_NW = _NC * _NS
_TPW = _B // _NW
_T = 8
_NCH = _TPW // _T
_P = _T * _NBR
_PPAD = _P + _L
_N3 = _P * _NEI
_N3PAD = -(-_N3 // _L) * _L
_NG = _N3PAD // _L
_NW2 = 2 * _N3PAD
_K2 = _NW2 // 128

_NVR = (_NBR + _L - 1) // _L


def _body(idx_hbm, eps_hbm, w_hbm, samp_hbm, logq_hbm,
          idxr_v, idx2_v, rows_v, eps_v, samp_v, logq_v, sem):
  c = lax.axis_index("c")
  s = lax.axis_index("s")
  wid = s * _NC + c

  iot = lax.iota(jnp.int32, _L)

  def chunk(ch, carry):
    tree0 = wid * _TPW + ch * _T
    pbase = tree0 * _NBR
    ibase = pbase * _NEI

    idxr_v[pl.ds(_N3PAD - _L, _L)] = jnp.zeros((_L,), jnp.int32)
    pltpu.sync_copy(idx_hbm.at[pl.ds(ibase, _N3)], idxr_v.at[pl.ds(0, _N3)])
    pltpu.sync_copy(eps_hbm.at[pl.ds(pbase, _P)], eps_v.at[pl.ds(0, _P)])

    def expand(g, cc):
      v = idxr_v[pl.ds(g * _L, _L)]
      v2 = v * 2
      base2 = g * (2 * _L) + 2 * iot
      plsc.store_scatter(idx2_v, [base2], v2)
      plsc.store_scatter(idx2_v, [base2 + 1], v2 + 1)
      return cc
    lax.fori_loop(0, _NG, expand, 0)

    def fire(j, cc):
      pltpu.async_copy(w_hbm.at[idx2_v.at[pl.ds(j * 128, 128)]],
                       rows_v.at[pl.ds(j * 128, 128)], sem)
      return cc
    lax.fori_loop(0, _K2, fire, 0)

    def drain(j, cc):
      pltpu.make_async_copy(w_hbm.at[idx2_v.at[pl.ds(j * 128, 128)]],
                            rows_v.at[pl.ds(j * 128, 128)], sem).wait()
      return cc
    lax.fori_loop(0, _K2, drain, 0)

    def tree(t, cc):
      base = t * _NBR
      acc = jnp.zeros((_L,), jnp.float32)
      for j in range(_NVR):
        off = base + j * _L
        pos = jnp.minimum(off + iot, _P - 1)
        f6 = pos * (2 * _NEI)
        m = (plsc.load_gather(rows_v, [f6])
             + plsc.load_gather(rows_v, [f6 + 2])
             + plsc.load_gather(rows_v, [f6 + 4]))
        sd = (plsc.load_gather(rows_v, [f6 + 1])
              + plsc.load_gather(rows_v, [f6 + 3])
              + plsc.load_gather(rows_v, [f6 + 5]))
        e = eps_v[pl.ds(off, _L)]
        samp_v[pl.ds(off, _L)] = e * jnp.exp(sd) + m
        valid = iot < (_NBR - j * _L)
        acc = acc + jnp.where(valid, -0.5 * e * e - sd, 0.0)
      tot = jnp.sum(acc) - 0.5 * _NBR * _LOG_2PI
      plsc.store_scatter(logq_v,
                         [jnp.broadcast_to(ch * _T + t, (_L,)).astype(jnp.int32)],
                         jnp.broadcast_to(tot, (_L,)),
                         mask=iot == 0)
      return cc
    lax.fori_loop(0, _T, tree, 0)

    pltpu.sync_copy(samp_v.at[pl.ds(0, _P)], samp_hbm.at[pl.ds(pbase, _P)])
    return carry

  lax.fori_loop(0, _NCH, chunk, 0)
  pltpu.sync_copy(logq_v, logq_hbm.at[pl.ds(wid * _TPW, _TPW)])


_encoder = functools.partial(
    pl.kernel,
    out_type=[jax.ShapeDtypeStruct((_B * _NBR,), jnp.float32),
              jax.ShapeDtypeStruct((_B,), jnp.float32)],
    mesh=plsc.VectorSubcoreMesh(core_axis_name="c", subcore_axis_name="s"),
    compiler_params=pltpu.CompilerParams(
        needs_layout_passes=False, use_tc_tiling_on_sc=False
    ),
    scratch_types=[
        pltpu.VMEM((_N3PAD,), jnp.int32),
        pltpu.VMEM((_NW2,), jnp.int32),
        pltpu.VMEM((_NW2,), jnp.float32),
        pltpu.VMEM((_PPAD,), jnp.float32),
        pltpu.VMEM((_PPAD,), jnp.float32),
        pltpu.VMEM((_TPW,), jnp.float32),
        pltpu.SemaphoreType.DMA,
    ],
)(_body)


@jax.jit
def kernel(neigh_ss_idxes, eps, W):
  idx_flat = neigh_ss_idxes.reshape(-1)
  eps_flat = eps.reshape(-1)
  w_flat = W.reshape(-1)
  samp_flat, logq = _encoder(idx_flat, eps_flat, w_flat)
  return samp_flat.reshape(_B, _NBR), logq, neigh_ss_idxes


# speedup vs baseline: 10.8508x; 1.0004x over previous
import functools

import jax
import jax.numpy as jnp
import numpy as np
from jax import lax
from jax.experimental import pallas as pl
from jax.experimental.pallas import tpu as pltpu
from jax.experimental.pallas import tpu_sc as plsc

_B = 4096
_NBR = 197
_NEI = 3
_LOG_2PI = float(np.log(2.0 * np.pi))

_L = 16
_NC, _NS = 2, 16
_NW = _NC * _NS
_TPW = _B // _NW
_T = 8
_NCH = _TPW // _T
_P = _T * _NBR
_PPAD = _P + _L
_N3 = _P * _NEI
_N3PAD = -(-_N3 // _L) * _L
_NG = _N3PAD // _L
_NW2 = 2 * _N3PAD
_K2 = _NW2 // 128

_NVR = (_NBR + _L - 1) // _L


def _body(idx_hbm, eps_hbm, w_hbm, samp_hbm, logq_hbm,
          idxr_v, idx2_v, rows_v, eps_v, samp_v, logq_v, sem):
  c = lax.axis_index("c")
  s = lax.axis_index("s")
  wid = s * _NC + c

  iot = lax.iota(jnp.int32, _L)

  def chunk(ch, carry):
    tree0 = wid * _TPW + ch * _T
    pbase = tree0 * _NBR
    ibase = pbase * _NEI

    idxr_v[pl.ds(_N3PAD - _L, _L)] = jnp.zeros((_L,), jnp.int32)
    pltpu.sync_copy(idx_hbm.at[pl.ds(ibase, _N3)], idxr_v.at[pl.ds(0, _N3)])
    pltpu.sync_copy(eps_hbm.at[pl.ds(pbase, _P)], eps_v.at[pl.ds(0, _P)])

    def expand(g, cc):
      v = idxr_v[pl.ds(g * _L, _L)]
      v2 = v * 2
      base2 = g * (2 * _L) + 2 * iot
      plsc.store_scatter(idx2_v, [base2], v2)
      plsc.store_scatter(idx2_v, [base2 + 1], v2 + 1)
      return cc
    lax.fori_loop(0, _NG, expand, 0)

    def fire(j, cc):
      pltpu.async_copy(w_hbm.at[idx2_v.at[pl.ds(j * 128, 128)]],
                       rows_v.at[pl.ds(j * 128, 128)], sem)
      return cc
    lax.fori_loop(0, _K2, fire, 0)

    def drain(j, cc):
      pltpu.make_async_copy(w_hbm.at[idx2_v.at[pl.ds(j * 128, 128)]],
                            rows_v.at[pl.ds(j * 128, 128)], sem).wait()
      return cc
    lax.fori_loop(0, _K2, drain, 0)

    def tree(t, cc):
      base = t * _NBR
      acc = jnp.zeros((_L,), jnp.float32)
      for j in range(_NVR):
        off = base + j * _L
        pos = jnp.minimum(off + iot, _P - 1)
        f6 = pos * (2 * _NEI)
        m = (plsc.load_gather(rows_v, [f6])
             + plsc.load_gather(rows_v, [f6 + 2])
             + plsc.load_gather(rows_v, [f6 + 4]))
        sd = (plsc.load_gather(rows_v, [f6 + 1])
              + plsc.load_gather(rows_v, [f6 + 3])
              + plsc.load_gather(rows_v, [f6 + 5]))
        e = eps_v[pl.ds(off, _L)]
        samp_v[pl.ds(off, _L)] = e * jnp.exp(sd) + m
        valid = iot < (_NBR - j * _L)
        acc = acc + jnp.where(valid, -0.5 * e * e - sd, 0.0)
      tot = jnp.sum(acc) - 0.5 * _NBR * _LOG_2PI
      plsc.store_scatter(logq_v,
                         [jnp.broadcast_to(ch * _T + t, (_L,)).astype(jnp.int32)],
                         jnp.broadcast_to(tot, (_L,)),
                         mask=iot == 0)
      return cc
    lax.fori_loop(0, _T, tree, 0)

    pltpu.sync_copy(samp_v.at[pl.ds(0, _P)], samp_hbm.at[pl.ds(pbase, _P)])
    return carry

  lax.fori_loop(0, _NCH, chunk, 0)
  pltpu.sync_copy(logq_v, logq_hbm.at[pl.ds(wid * _TPW, _TPW)])


_encoder = functools.partial(
    pl.kernel,
    out_type=[jax.ShapeDtypeStruct((_B * _NBR,), jnp.float32),
              jax.ShapeDtypeStruct((_B,), jnp.float32)],
    mesh=plsc.VectorSubcoreMesh(core_axis_name="c", subcore_axis_name="s"),
    compiler_params=pltpu.CompilerParams(
        needs_layout_passes=False, use_tc_tiling_on_sc=True
    ),
    scratch_types=[
        pltpu.VMEM((_N3PAD,), jnp.int32),
        pltpu.VMEM((_NW2,), jnp.int32),
        pltpu.VMEM((_NW2,), jnp.float32),
        pltpu.VMEM((_PPAD,), jnp.float32),
        pltpu.VMEM((_PPAD,), jnp.float32),
        pltpu.VMEM((_TPW,), jnp.float32),
        pltpu.SemaphoreType.DMA,
    ],
)(_body)


@jax.jit
def kernel(neigh_ss_idxes, eps, W):
  idx_flat = neigh_ss_idxes.reshape(-1)
  eps_flat = eps.reshape(-1)
  w_flat = W.reshape(-1)
  samp_flat, logq = _encoder(idx_flat, eps_flat, w_flat)
  return samp_flat.reshape(_B, _NBR), logq, neigh_ss_idxes


# speedup vs baseline: 144.7961x; 13.3442x over previous
import functools

import jax
import jax.numpy as jnp
import numpy as np
from jax import lax
from jax.experimental import pallas as pl
from jax.experimental.pallas import tpu as pltpu
from jax.experimental.pallas import tpu_sc as plsc

_B = 4096
_NBR = 197
_NEI = 3
_LOG_2PI = float(np.log(2.0 * np.pi))

_L = 16
_NC, _NS = 2, 16
_NW = _NC * _NS
_TPW = _B // _NW
_TV = _TPW // _L

_BR = 32
_NCH = _NBR // _BR
_BRT = _NBR - _NCH * _BR
_SLAB = _NEI * _BR * _TPW


def _body(idx_hbm, eps_hbm, wm_hbm, ws_hbm, samp_hbm, logq_hbm,
          idx3_v, idx1_v, rows_m, rows_s, eps_v, samp_v, acc_v, sem):
  c = lax.axis_index("c")
  s = lax.axis_index("s")
  wid = s * _NC + c
  tbase = wid * _TPW

  for k in range(_TV):
    acc_v[pl.ds(k * _L, _L)] = jnp.zeros((_L,), jnp.float32)

  def do_chunk(br0, br_n):
    slab = _NEI * br_n * _TPW
    pltpu.sync_copy(
        idx_hbm.at[:, pl.ds(br0, br_n), pl.ds(tbase, _TPW)],
        idx3_v.at[:, pl.ds(0, br_n)])
    pltpu.sync_copy(
        eps_hbm.at[pl.ds(br0, br_n), pl.ds(tbase, _TPW)],
        eps_v.at[pl.ds(0, br_n)])

    def flat_br(n):
      def inner(br, cc):
        r = n * br_n + br
        for k in range(_TV):
          idx1_v[pl.ds(r * _TPW + k * _L, _L)] = idx3_v[n, br, pl.ds(k * _L, _L)]
        return cc
      return inner
    for n in range(_NEI):
      lax.fori_loop(0, br_n, flat_br(n), 0)

    pltpu.async_copy(wm_hbm.at[idx1_v.at[pl.ds(0, slab)]],
                     rows_m.at[pl.ds(0, slab)], sem)
    pltpu.async_copy(ws_hbm.at[idx1_v.at[pl.ds(0, slab)]],
                     rows_s.at[pl.ds(0, slab)], sem)
    pltpu.make_async_copy(wm_hbm.at[idx1_v.at[pl.ds(0, slab)]],
                          rows_m.at[pl.ds(0, slab)], sem).wait()
    pltpu.make_async_copy(ws_hbm.at[idx1_v.at[pl.ds(0, slab)]],
                          rows_s.at[pl.ds(0, slab)], sem).wait()

    def comp_row(br, cc):
      o0 = br * _TPW
      o1 = (br_n + br) * _TPW
      o2 = (2 * br_n + br) * _TPW
      for k in range(_TV):
        kk = k * _L
        m = (rows_m[pl.ds(o0 + kk, _L)] + rows_m[pl.ds(o1 + kk, _L)]
             + rows_m[pl.ds(o2 + kk, _L)])
        sd = (rows_s[pl.ds(o0 + kk, _L)] + rows_s[pl.ds(o1 + kk, _L)]
              + rows_s[pl.ds(o2 + kk, _L)])
        e = eps_v[br, pl.ds(kk, _L)]
        samp_v[br, pl.ds(kk, _L)] = e * jnp.exp(sd) + m
        acc_v[pl.ds(kk, _L)] = acc_v[pl.ds(kk, _L)] - 0.5 * e * e - sd
      return cc
    lax.fori_loop(0, br_n, comp_row, 0)

    pltpu.sync_copy(
        samp_v.at[pl.ds(0, br_n)],
        samp_hbm.at[pl.ds(br0, br_n), pl.ds(tbase, _TPW)])

  def chunk(ch, carry):
    do_chunk(ch * _BR, _BR)
    return carry
  lax.fori_loop(0, _NCH, chunk, 0)
  do_chunk(_NCH * _BR, _BRT)

  for k in range(_TV):
    acc_v[pl.ds(k * _L, _L)] = (
        acc_v[pl.ds(k * _L, _L)] - 0.5 * _NBR * _LOG_2PI)
  pltpu.sync_copy(acc_v, logq_hbm.at[pl.ds(tbase, _TPW)])


_encoder = functools.partial(
    pl.kernel,
    out_type=[jax.ShapeDtypeStruct((_NBR, _B), jnp.float32),
              jax.ShapeDtypeStruct((_B,), jnp.float32)],
    mesh=plsc.VectorSubcoreMesh(core_axis_name="c", subcore_axis_name="s"),
    compiler_params=pltpu.CompilerParams(
        needs_layout_passes=False, use_tc_tiling_on_sc=True
    ),
    scratch_types=[
        pltpu.VMEM((_NEI, _BR, _TPW), jnp.int32),
        pltpu.VMEM((_SLAB,), jnp.int32),
        pltpu.VMEM((_SLAB,), jnp.float32),
        pltpu.VMEM((_SLAB,), jnp.float32),
        pltpu.VMEM((_BR, _TPW), jnp.float32),
        pltpu.VMEM((_BR, _TPW), jnp.float32),
        pltpu.VMEM((_TPW,), jnp.float32),
        pltpu.SemaphoreType.DMA,
    ],
)(_body)


@jax.jit
def kernel(neigh_ss_idxes, eps, W):
  idx_t = jnp.transpose(neigh_ss_idxes, (2, 1, 0))
  eps_t = eps.T
  w_mean = W[:, 0]
  w_std = W[:, 1]
  samp_t, logq = _encoder(idx_t, eps_t, w_mean, w_std)
  return samp_t.T, logq, neigh_ss_idxes


# speedup vs baseline: 162.8163x; 1.1245x over previous
import functools

import jax
import jax.numpy as jnp
import numpy as np
from jax import lax
from jax.experimental import pallas as pl
from jax.experimental.pallas import tpu as pltpu
from jax.experimental.pallas import tpu_sc as plsc

_B = 4096
_NBR = 197
_NEI = 3
_LOG_2PI = float(np.log(2.0 * np.pi))

_L = 16
_NC, _NS = 2, 16
_NW = _NC * _NS
_TPW = _B // _NW
_TV = _TPW // _L

_BR = 32
_NCH = _NBR // _BR
_BRT = _NBR - _NCH * _BR
_SLAB = _NEI * _BR * _TPW

_CHUNKS = [(i * _BR, _BR) for i in range(_NCH)] + [(_NCH * _BR, _BRT)]


def _body(idx_hbm, eps_hbm, wm_hbm, ws_hbm, samp_hbm, logq_hbm,
          idx3_v, idx1_v0, idx1_v1, rows_m0, rows_m1, rows_s0, rows_s1,
          eps_v, samp_v, acc_v,
          sem_stage, sem_rows0, sem_rows1):
  c = lax.axis_index("c")
  s = lax.axis_index("s")
  wid = s * _NC + c
  tbase = wid * _TPW

  for k in range(_TV):
    acc_v[pl.ds(k * _L, _L)] = jnp.zeros((_L,), jnp.float32)

  def stage_start(g):
    br0, br_n = _CHUNKS[g]
    b = g % 2
    pltpu.async_copy(
        idx_hbm.at[:, pl.ds(br0, br_n), pl.ds(tbase, _TPW)],
        idx3_v.at[b, :, pl.ds(0, br_n)], sem_stage)
    pltpu.async_copy(
        eps_hbm.at[pl.ds(br0, br_n), pl.ds(tbase, _TPW)],
        eps_v.at[b, pl.ds(0, br_n)], sem_stage)

  def stage_wait(g):
    br0, br_n = _CHUNKS[g]
    b = g % 2
    pltpu.make_async_copy(
        idx_hbm.at[:, pl.ds(br0, br_n), pl.ds(tbase, _TPW)],
        idx3_v.at[b, :, pl.ds(0, br_n)], sem_stage).wait()
    pltpu.make_async_copy(
        eps_hbm.at[pl.ds(br0, br_n), pl.ds(tbase, _TPW)],
        eps_v.at[b, pl.ds(0, br_n)], sem_stage).wait()

  def flatten_and_fire(g):
    _, br_n = _CHUNKS[g]
    b = g % 2
    slab = _NEI * br_n * _TPW
    idx1_v = idx1_v0 if b == 0 else idx1_v1
    rows_m = rows_m0 if b == 0 else rows_m1
    rows_s = rows_s0 if b == 0 else rows_s1

    def flat_br(n):
      def inner(br, cc):
        r = n * br_n + br
        for k in range(_TV):
          idx1_v[pl.ds(r * _TPW + k * _L, _L)] = (
              idx3_v[b, n, br, pl.ds(k * _L, _L)])
        return cc
      return inner
    for n in range(_NEI):
      lax.fori_loop(0, br_n, flat_br(n), 0)

    sem_rows = sem_rows0 if b == 0 else sem_rows1
    pltpu.async_copy(wm_hbm.at[idx1_v.at[pl.ds(0, slab)]],
                     rows_m.at[pl.ds(0, slab)], sem_rows)
    pltpu.async_copy(ws_hbm.at[idx1_v.at[pl.ds(0, slab)]],
                     rows_s.at[pl.ds(0, slab)], sem_rows)

  def drain_and_compute(g):
    br0, br_n = _CHUNKS[g]
    b = g % 2
    slab = _NEI * br_n * _TPW
    sem_rows = sem_rows0 if b == 0 else sem_rows1
    idx1_v = idx1_v0 if b == 0 else idx1_v1
    rows_m = rows_m0 if b == 0 else rows_m1
    rows_s = rows_s0 if b == 0 else rows_s1
    pltpu.make_async_copy(wm_hbm.at[idx1_v.at[pl.ds(0, slab)]],
                          rows_m.at[pl.ds(0, slab)], sem_rows).wait()
    pltpu.make_async_copy(ws_hbm.at[idx1_v.at[pl.ds(0, slab)]],
                          rows_s.at[pl.ds(0, slab)], sem_rows).wait()

    def comp_row(br, cc):
      o0 = br * _TPW
      o1 = (br_n + br) * _TPW
      o2 = (2 * br_n + br) * _TPW
      for k in range(_TV):
        kk = k * _L
        m = (rows_m[pl.ds(o0 + kk, _L)] + rows_m[pl.ds(o1 + kk, _L)]
             + rows_m[pl.ds(o2 + kk, _L)])
        sd = (rows_s[pl.ds(o0 + kk, _L)] + rows_s[pl.ds(o1 + kk, _L)]
              + rows_s[pl.ds(o2 + kk, _L)])
        e = eps_v[b, br, pl.ds(kk, _L)]
        samp_v[b, br, pl.ds(kk, _L)] = e * jnp.exp(sd) + m
        acc_v[pl.ds(kk, _L)] = acc_v[pl.ds(kk, _L)] - 0.5 * e * e - sd
      return cc
    lax.fori_loop(0, br_n, comp_row, 0)

    pltpu.sync_copy(
        samp_v.at[b, pl.ds(0, br_n)],
        samp_hbm.at[pl.ds(br0, br_n), pl.ds(tbase, _TPW)])

  n_chunks = len(_CHUNKS)
  stage_start(0)
  stage_wait(0)
  flatten_and_fire(0)
  for g in range(n_chunks):
    if g + 1 < n_chunks:
      stage_start(g + 1)
      stage_wait(g + 1)
      flatten_and_fire(g + 1)
    drain_and_compute(g)

  for k in range(_TV):
    acc_v[pl.ds(k * _L, _L)] = (
        acc_v[pl.ds(k * _L, _L)] - 0.5 * _NBR * _LOG_2PI)
  pltpu.sync_copy(acc_v, logq_hbm.at[pl.ds(tbase, _TPW)])


_encoder = functools.partial(
    pl.kernel,
    out_type=[jax.ShapeDtypeStruct((_NBR, _B), jnp.float32),
              jax.ShapeDtypeStruct((_B,), jnp.float32)],
    mesh=plsc.VectorSubcoreMesh(core_axis_name="c", subcore_axis_name="s"),
    compiler_params=pltpu.CompilerParams(
        needs_layout_passes=False, use_tc_tiling_on_sc=True
    ),
    scratch_types=[
        pltpu.VMEM((2, _NEI, _BR, _TPW), jnp.int32),
        pltpu.VMEM((_SLAB,), jnp.int32),
        pltpu.VMEM((_SLAB,), jnp.int32),
        pltpu.VMEM((_SLAB,), jnp.float32),
        pltpu.VMEM((_SLAB,), jnp.float32),
        pltpu.VMEM((_SLAB,), jnp.float32),
        pltpu.VMEM((_SLAB,), jnp.float32),
        pltpu.VMEM((2, _BR, _TPW), jnp.float32),
        pltpu.VMEM((2, _BR, _TPW), jnp.float32),
        pltpu.VMEM((_TPW,), jnp.float32),
        pltpu.SemaphoreType.DMA,
        pltpu.SemaphoreType.DMA,
        pltpu.SemaphoreType.DMA,
    ],
)(_body)


@jax.jit
def kernel(neigh_ss_idxes, eps, W):
  idx_t = jnp.transpose(neigh_ss_idxes, (2, 1, 0))
  eps_t = eps.T
  w_mean = W[:, 0]
  w_std = W[:, 1]
  samp_t, logq = _encoder(idx_t, eps_t, w_mean, w_std)
  return samp_t.T, logq, neigh_ss_idxes
